# baseline (speedup 1.0000x reference)
import functools

import jax
import jax.numpy as jnp
from jax import lax
from jax.experimental import pallas as pl
from jax.experimental.pallas import tpu as pltpu
from jax.experimental.pallas import tpu_sc as plsc

N = 10000
R = 16
NT = 160000
RP = 16
EMB = 32
C = 16
E = RP * NT
NSEG = N * RP
NC = 2
NS = 16
NW = NC * NS
EPT = E // NW
i32 = jnp.int32
f32 = jnp.float32


def _mesh():
    return plsc.VectorSubcoreMesh(
        core_axis_name="c", subcore_axis_name="s", num_cores=NC, num_subcores=NS
    )


def _zero_fill(ref, n):
    z = jnp.zeros((16,), f32)

    def body(i, _):
        ref[pl.ds(i * 16, 16)] = z
        return 0

    lax.fori_loop(0, n // 16, body, 0)


def _zero_fill2d(ref, nrows, ncols):
    z = jnp.zeros((16,), f32)
    cpr = ncols // 16

    def body(i, _):
        ref[i // cpr, pl.ds((i % cpr) * 16, 16)] = z
        return 0

    lax.fori_loop(0, nrows * cpr, body, 0)


def _wid_r_t0():
    cid = lax.axis_index("c")
    sid = lax.axis_index("s")
    wid = cid * NS + sid
    return cid, sid, wid // 2, (wid % 2) * EPT


def _prep_body(nhots_ref, wl1_ref, bl1_ref, wl2_ref, bl2_ref,
               p_ref, t1_ref, t2_ref):
    pid = pl.program_id(0)
    nh = nhots_ref[...]
    ridx = lax.broadcasted_iota(i32, nh.shape, 1).astype(f32)
    p_ref[...] = jnp.sum(nh * ridx, axis=1, keepdims=True).astype(i32)

    @pl.when(pid == 0)
    def _():
        for w_ref, b_ref, t_ref in ((wl1_ref, bl1_ref, t1_ref),
                                    (wl2_ref, bl2_ref, t2_ref)):
            z = w_ref[...] + b_ref[...]
            z = z - jnp.max(z, axis=1, keepdims=True)
            ez = jnp.exp(z)
            t_ref[...] = ez / jnp.sum(ez, axis=1, keepdims=True)


def _prep(nhots, wl1, bl1, wl2, bl2):
    blk = 4000
    return pl.pallas_call(
        _prep_body,
        grid=(NT // blk,),
        in_specs=[
            pl.BlockSpec((blk, R), lambda i: (i, 0)),
            pl.BlockSpec((R, RP), lambda i: (0, 0)),
            pl.BlockSpec((1, RP), lambda i: (0, 0)),
            pl.BlockSpec((R, RP), lambda i: (0, 0)),
            pl.BlockSpec((1, RP), lambda i: (0, 0)),
        ],
        out_specs=[
            pl.BlockSpec((blk, 1), lambda i: (i, 0)),
            pl.BlockSpec((R, RP), lambda i: (0, 0)),
            pl.BlockSpec((R, RP), lambda i: (0, 0)),
        ],
        out_shape=[
            jax.ShapeDtypeStruct((NT, 1), i32),
            jax.ShapeDtypeStruct((R, RP), f32),
            jax.ShapeDtypeStruct((R, RP), f32),
        ],
    )(nhots, wl1, bl1, wl2, bl2)


_AB = 3200


def _sums_body(s_hbm, o_hbm, p_hbm, t1_hbm, t2_hbm,
               colsum_hbm, rowsum_hbm,
               csum_sp, rsum_sp, t1v, t2v, sbuf, obuf, pbuf,
               colv, rowv, v1v, v2v, zb, sem):
    cid, sid, r, t0 = _wid_r_t0()
    B = _AB
    KB = B // 128

    _zero_fill(zb, 2000)

    def zbody(i, _):
        pltpu.sync_copy(zb, csum_sp.at[pl.ds(sid * 10000 + i * 2000, 2000)])
        pltpu.sync_copy(zb, rsum_sp.at[pl.ds(sid * 10000 + i * 2000, 2000)])
        return 0

    lax.fori_loop(0, 5, zbody, 0)
    pltpu.sync_copy(t1_hbm, t1v)
    pltpu.sync_copy(t2_hbm, t2v)
    plsc.subcore_barrier()

    @pl.when(r > 0)
    def _():
        def block(blk, _):
            tb = t0 + blk * B
            pltpu.sync_copy(s_hbm.at[pl.ds(tb, B)], sbuf)
            pltpu.sync_copy(o_hbm.at[pl.ds(tb, B)], obuf)
            pltpu.sync_copy(p_hbm.at[pl.ds(tb, B)], pbuf)

            @plsc.parallel_loop(0, B // 16)
            def inner(g):
                sv = sbuf[pl.ds(g * 16, 16)]
                ov = obuf[pl.ds(g * 16, 16)]
                pv = pbuf[pl.ds(g * 16, 16)]
                j, off = g // 8, (g % 8) * 16
                colv[j, pl.ds(off, 16)] = ov * r
                rowv[j, pl.ds(off, 16)] = sv * r
                tidx = pv * RP + r
                v1v[j, pl.ds(off, 16)] = plsc.load_gather(t1v, [tidx])
                v2v[j, pl.ds(off, 16)] = plsc.load_gather(t2v, [tidx])

            descs = [pltpu.async_copy(v1v.at[j], csum_sp.at[colv.at[j]], sem,
                                      add=True) for j in range(KB)]
            for d in descs:
                d.wait()
            descs = [pltpu.async_copy(v2v.at[j], rsum_sp.at[rowv.at[j]], sem,
                                      add=True) for j in range(KB)]
            for d in descs:
                d.wait()
            return 0

        lax.fori_loop(0, EPT // B, block, 0)

    @pl.when(r == 0)
    def _():
        def block(blk, acc):
            tb = t0 + blk * B
            pltpu.sync_copy(p_hbm.at[pl.ds(tb, B)], pbuf)

            def inner(g, acc2):
                a1, a2 = acc2
                pv = pbuf[pl.ds(g * 16, 16)]
                a1 = a1 + plsc.load_gather(t1v, [pv * RP])
                a2 = a2 + plsc.load_gather(t2v, [pv * RP])
                return a1, a2

            return lax.fori_loop(0, B // 16, inner, acc)

        z16 = jnp.zeros((16,), f32)
        zi16 = jnp.zeros((16,), i32)
        a1, a2 = lax.fori_loop(0, EPT // B, block, (z16, z16))
        for k in range(8):
            v1v[0, pl.ds(k * 16, 16)] = z16
            v2v[0, pl.ds(k * 16, 16)] = z16
            colv[0, pl.ds(k * 16, 16)] = zi16
            rowv[0, pl.ds(k * 16, 16)] = zi16
        v1v[0, pl.ds(0, 16)] = a1
        v2v[0, pl.ds(0, 16)] = a2
        pltpu.async_copy(v1v.at[0], csum_sp.at[colv.at[0]], sem,
                         add=True).wait()
        pltpu.async_copy(v2v.at[0], rsum_sp.at[rowv.at[0]], sem,
                         add=True).wait()

    plsc.subcore_barrier()

    @pl.when(sid == 0)
    def _():
        pltpu.sync_copy(csum_sp, colsum_hbm.at[cid])
        pltpu.sync_copy(rsum_sp, rowsum_hbm.at[cid])


def _sums(s, o, p, t1f, t2f):
    B = _AB
    return pl.kernel(
        _sums_body,
        out_type=[
            jax.ShapeDtypeStruct((NC, NSEG), f32),
            jax.ShapeDtypeStruct((NC, NSEG), f32),
        ],
        mesh=_mesh(),
        compiler_params=pltpu.CompilerParams(needs_layout_passes=False, use_tc_tiling_on_sc=False),
        scratch_types=[
            pltpu.VMEM_SHARED((NSEG,), f32),
            pltpu.VMEM_SHARED((NSEG,), f32),
            pltpu.VMEM((R * RP,), f32),
            pltpu.VMEM((R * RP,), f32),
            pltpu.VMEM((B,), i32),
            pltpu.VMEM((B,), i32),
            pltpu.VMEM((B,), i32),
            pltpu.VMEM((B // 128, 128), i32),
            pltpu.VMEM((B // 128, 128), i32),
            pltpu.VMEM((B // 128, 128), f32),
            pltpu.VMEM((B // 128, 128), f32),
            pltpu.VMEM((2000,), f32),
            pltpu.SemaphoreType.DMA,
        ],
    )(s, o, p, t1f, t2f)


def _norm_body(w1_ref, cs_ref, rs_ref, w1n_ref, rrec_ref):
    cs = cs_ref[0] + cs_ref[1]
    rs = rs_ref[0] + rs_ref[1]
    w1n_ref[...] = w1_ref[...] / cs
    rrec_ref[...] = 1.0 / rs


def _norm(w1flat, colsum, rowsum):
    blk = 3200
    return pl.pallas_call(
        _norm_body,
        grid=(NSEG // blk,),
        in_specs=[
            pl.BlockSpec((blk, EMB), lambda i: (i, 0)),
            pl.BlockSpec((NC, blk, 1), lambda i: (0, i, 0)),
            pl.BlockSpec((NC, blk, 1), lambda i: (0, i, 0)),
        ],
        out_specs=[
            pl.BlockSpec((blk, EMB), lambda i: (i, 0)),
            pl.BlockSpec((blk, 1), lambda i: (i, 0)),
        ],
        out_shape=[
            jax.ShapeDtypeStruct((NSEG, EMB), f32),
            jax.ShapeDtypeStruct((NSEG, 1), f32),
        ],
    )(w1flat, colsum.reshape(NC, NSEG, 1), rowsum.reshape(NC, NSEG, 1))


_CB = 640


def _pass1_body(s_hbm, o_hbm, p_hbm, t1_hbm, w1n_hbm, hp_hbm,
                h_sp, t1v, sbuf, obuf, pbuf, gidxv, dstv, facv, rows,
                zrows, w1r0, isem, gsem, ssem):
    cid, sid, r, t0 = _wid_r_t0()
    B = _CB
    KB = B // 128

    _zero_fill2d(zrows, 125, EMB)

    def zbody(i, _):
        pltpu.sync_copy(zrows, h_sp.at[pl.ds(sid * 625 + i * 125, 125), :])
        return 0

    lax.fori_loop(0, 5, zbody, 0)
    pltpu.sync_copy(t1_hbm, t1v)
    pltpu.sync_copy(w1n_hbm.at[0], w1r0)
    w0a = w1r0[pl.ds(0, 16)]
    w0b = w1r0[pl.ds(16, 16)]
    plsc.subcore_barrier()

    NBLK = EPT // B

    def fire_inputs(blk):
        st = blk & 1
        tb = t0 + blk * B
        pltpu.async_copy(s_hbm.at[pl.ds(tb, B)], sbuf.at[st], isem)
        pltpu.async_copy(o_hbm.at[pl.ds(tb, B)], obuf.at[st], isem)
        pltpu.async_copy(p_hbm.at[pl.ds(tb, B)], pbuf.at[st], isem)

    def wait_inputs(blk):
        st = blk & 1
        tb = t0 + blk * B
        pltpu.make_async_copy(s_hbm.at[pl.ds(tb, B)], sbuf.at[st], isem).wait()
        pltpu.make_async_copy(o_hbm.at[pl.ds(tb, B)], obuf.at[st], isem).wait()
        pltpu.make_async_copy(p_hbm.at[pl.ds(tb, B)], pbuf.at[st], isem).wait()

    def index_compute(st):
        @plsc.parallel_loop(0, B // 16)
        def inner(g):
            sv = sbuf[st, pl.ds(g * 16, 16)]
            ov = obuf[st, pl.ds(g * 16, 16)]
            pv = pbuf[st, pl.ds(g * 16, 16)]
            gidxv[st, pl.ds(g * 16, 16)] = ov * r
            j, off = g // 8, (g % 8) * 16
            dstv[st, j, pl.ds(off, 16)] = sv
            facv[st, pl.ds(g * 16, 16)] = plsc.load_gather(t1v, [pv * RP + r])

    def fire_gather(st):
        @pl.when(r > 0)
        def _():
            pltpu.async_copy(w1n_hbm.at[gidxv.at[st]], rows.at[st], gsem)

    def wait_gather(st):
        @pl.when(r > 0)
        def _():
            pltpu.make_async_copy(w1n_hbm.at[gidxv.at[st]], rows.at[st],
                                  gsem).wait()

    def scale(st):
        @pl.when(r > 0)
        def _():
            @plsc.parallel_loop(0, B // 16)
            def body(g):
                for b2 in range(16):
                    e = g * 16 + b2
                    fv = plsc.load_gather(facv.at[st],
                                          [jnp.full((16,), 0, i32) + e])
                    rows[st, e, pl.ds(0, 16)] = rows[st, e, pl.ds(0, 16)] * fv
                    rows[st, e, pl.ds(16, 16)] = rows[st, e, pl.ds(16, 16)] * fv

        @pl.when(r == 0)
        def _():
            @plsc.parallel_loop(0, B // 16)
            def body0(g):
                for b2 in range(16):
                    e = g * 16 + b2
                    fv = plsc.load_gather(facv.at[st],
                                          [jnp.full((16,), 0, i32) + e])
                    rows[st, e, pl.ds(0, 16)] = w0a * fv
                    rows[st, e, pl.ds(16, 16)] = w0b * fv

    def fire_scatters(st):
        for j in range(KB):
            pltpu.async_copy(rows.at[st, pl.ds(j * 128, 128), :],
                             h_sp.at[dstv.at[st, j]], ssem, add=True)

    def wait_scatters(st):
        for j in range(KB):
            pltpu.make_async_copy(rows.at[st, pl.ds(j * 128, 128), :],
                                  h_sp.at[dstv.at[st, j]], ssem).wait()

    fire_inputs(0)
    wait_inputs(0)
    index_compute(0)
    fire_gather(0)
    fire_inputs(1)

    def block(blk, _):
        st = blk & 1
        nxt = 1 - st
        wait_gather(st)
        scale(st)
        fire_scatters(st)

        @pl.when((blk > 0) & (blk < NBLK - 1))
        def _():
            wait_scatters(nxt)

        @pl.when(blk < NBLK - 2)
        def _():
            fire_inputs(blk + 2)

        @pl.when(blk < NBLK - 1)
        def _():
            wait_inputs(blk + 1)
            index_compute(nxt)
            fire_gather(nxt)

        return 0

    lax.fori_loop(0, NBLK, block, 0)
    wait_scatters((NBLK - 2) & 1)
    wait_scatters((NBLK - 1) & 1)
    plsc.subcore_barrier()

    @pl.when(sid == 0)
    def _():
        pltpu.sync_copy(h_sp, hp_hbm.at[cid])


def _pass1(s, o, p, t1f, w1n):
    B = _CB
    return pl.kernel(
        _pass1_body,
        out_type=jax.ShapeDtypeStruct((NC, N, EMB), f32),
        mesh=_mesh(),
        compiler_params=pltpu.CompilerParams(needs_layout_passes=False, use_tc_tiling_on_sc=False),
        scratch_types=[
            pltpu.VMEM_SHARED((N, EMB), f32),
            pltpu.VMEM((R * RP,), f32),
            pltpu.VMEM((2, B), i32),
            pltpu.VMEM((2, B), i32),
            pltpu.VMEM((2, B), i32),
            pltpu.VMEM((2, B), i32),
            pltpu.VMEM((2, B // 128, 128), i32),
            pltpu.VMEM((2, B), f32),
            pltpu.VMEM((2, B, EMB), f32),
            pltpu.VMEM((125, EMB), f32),
            pltpu.VMEM((EMB,), f32),
            pltpu.SemaphoreType.DMA,
            pltpu.SemaphoreType.DMA,
            pltpu.SemaphoreType.DMA,
        ],
    )(s, o, p, t1f, w1n)


def _g_body(hp_ref, w2_ref, b1_ref, g_ref):
    h = jnp.maximum(hp_ref[0] + hp_ref[1] + b1_ref[...], 0.0)
    g_ref[0] = jnp.dot(h, w2_ref[0], preferred_element_type=f32)


def _gtable(hp, w2, b1):
    blk = 2000
    return pl.pallas_call(
        _g_body,
        grid=(RP, N // blk),
        in_specs=[
            pl.BlockSpec((NC, blk, EMB), lambda r, i: (0, i, 0)),
            pl.BlockSpec((1, EMB, C), lambda r, i: (r, 0, 0)),
            pl.BlockSpec((1, EMB), lambda r, i: (0, 0)),
        ],
        out_specs=pl.BlockSpec((1, blk, C), lambda r, i: (r, i, 0)),
        out_shape=jax.ShapeDtypeStruct((RP, N, C), f32),
    )(hp, w2, b1)


def _pass2_body(s_hbm, o_hbm, p_hbm, t2_hbm, rrec_hbm, g_hbm, outp_hbm,
                out_sp, t2v, rrv, rstage, sbuf, obuf, pbuf, gidxv, dstv, facv,
                rows, zrows, isem, gsem, ssem):
    cid, sid, r, t0 = _wid_r_t0()
    B = _CB
    KB = B // 128

    _zero_fill2d(zrows, 125, C)

    def zbody(i, _):
        pltpu.sync_copy(zrows, out_sp.at[pl.ds(sid * 625 + i * 125, 125), :])
        return 0

    lax.fori_loop(0, 5, zbody, 0)
    pltpu.sync_copy(t2_hbm, t2v)

    lane = lax.iota(i32, 16)
    for q in range(5):
        pltpu.sync_copy(rrec_hbm.at[pl.ds(q * 2000 * r, 32000)], rstage)

        @plsc.parallel_loop(0, 125)
        def rbody(g):
            idx = (g * 16 + lane) * r
            rrv[pl.ds(q * 2000 + g * 16, 16)] = plsc.load_gather(rstage, [idx])

    plsc.subcore_barrier()

    NBLK = EPT // B

    def fire_inputs(blk):
        st = blk & 1
        tb = t0 + blk * B
        pltpu.async_copy(s_hbm.at[pl.ds(tb, B)], sbuf.at[st], isem)
        pltpu.async_copy(o_hbm.at[pl.ds(tb, B)], obuf.at[st], isem)
        pltpu.async_copy(p_hbm.at[pl.ds(tb, B)], pbuf.at[st], isem)

    def wait_inputs(blk):
        st = blk & 1
        tb = t0 + blk * B
        pltpu.make_async_copy(s_hbm.at[pl.ds(tb, B)], sbuf.at[st], isem).wait()
        pltpu.make_async_copy(o_hbm.at[pl.ds(tb, B)], obuf.at[st], isem).wait()
        pltpu.make_async_copy(p_hbm.at[pl.ds(tb, B)], pbuf.at[st], isem).wait()

    def index_compute(st):
        @plsc.parallel_loop(0, B // 16)
        def inner(g):
            sv = sbuf[st, pl.ds(g * 16, 16)]
            ov = obuf[st, pl.ds(g * 16, 16)]
            pv = pbuf[st, pl.ds(g * 16, 16)]
            rowi = sv * r
            rpv = ((rowi.astype(f32) + 0.5) * (1.0 / N)).astype(i32)
            gidxv[st, pl.ds(g * 16, 16)] = rpv * N + ov
            j, off = g // 8, (g % 8) * 16
            dstv[st, j, pl.ds(off, 16)] = rowi - rpv * N
            t2g = plsc.load_gather(t2v, [pv * RP + r])
            rrg = plsc.load_gather(rrv, [sv])
            facv[st, pl.ds(g * 16, 16)] = t2g * rrg

    def fire_gather(st):
        pltpu.async_copy(g_hbm.at[gidxv.at[st]], rows.at[st], gsem)

    def wait_gather(st):
        pltpu.make_async_copy(g_hbm.at[gidxv.at[st]], rows.at[st], gsem).wait()

    def scale(st):
        @plsc.parallel_loop(0, B // 16)
        def body(g):
            for b2 in range(16):
                e = g * 16 + b2
                fv = plsc.load_gather(facv.at[st],
                                      [jnp.full((16,), 0, i32) + e])
                rows[st, e, pl.ds(0, 16)] = rows[st, e, pl.ds(0, 16)] * fv

    def fire_scatters(st):
        for j in range(KB):
            pltpu.async_copy(rows.at[st, pl.ds(j * 128, 128), :],
                             out_sp.at[dstv.at[st, j]], ssem, add=True)

    def wait_scatters(st):
        for j in range(KB):
            pltpu.make_async_copy(rows.at[st, pl.ds(j * 128, 128), :],
                                  out_sp.at[dstv.at[st, j]], ssem).wait()

    fire_inputs(0)
    wait_inputs(0)
    index_compute(0)
    fire_gather(0)
    fire_inputs(1)

    def block(blk, _):
        st = blk & 1
        nxt = 1 - st
        wait_gather(st)
        scale(st)
        fire_scatters(st)

        @pl.when((blk > 0) & (blk < NBLK - 1))
        def _():
            wait_scatters(nxt)

        @pl.when(blk < NBLK - 2)
        def _():
            fire_inputs(blk + 2)

        @pl.when(blk < NBLK - 1)
        def _():
            wait_inputs(blk + 1)
            index_compute(nxt)
            fire_gather(nxt)

        return 0

    lax.fori_loop(0, NBLK, block, 0)
    wait_scatters((NBLK - 2) & 1)
    wait_scatters((NBLK - 1) & 1)
    plsc.subcore_barrier()

    @pl.when(sid == 0)
    def _():
        pltpu.sync_copy(out_sp, outp_hbm.at[cid])


def _pass2(s, o, p, t2f, rrec, gflat):
    B = _CB
    return pl.kernel(
        _pass2_body,
        out_type=jax.ShapeDtypeStruct((NC, N, C), f32),
        mesh=_mesh(),
        compiler_params=pltpu.CompilerParams(needs_layout_passes=False, use_tc_tiling_on_sc=False),
        scratch_types=[
            pltpu.VMEM_SHARED((N, C), f32),
            pltpu.VMEM((R * RP,), f32),
            pltpu.VMEM((N,), f32),
            pltpu.VMEM((32000,), f32),
            pltpu.VMEM((2, B), i32),
            pltpu.VMEM((2, B), i32),
            pltpu.VMEM((2, B), i32),
            pltpu.VMEM((2, B), i32),
            pltpu.VMEM((2, B // 128, 128), i32),
            pltpu.VMEM((2, B), f32),
            pltpu.VMEM((2, B, C), f32),
            pltpu.VMEM((125, C), f32),
            pltpu.SemaphoreType.DMA,
            pltpu.SemaphoreType.DMA,
            pltpu.SemaphoreType.DMA,
        ],
    )(s, o, p, t2f, rrec, gflat)


def _final_body(op_ref, b2_ref, out_ref):
    out_ref[...] = op_ref[0] + op_ref[1] + b2_ref[...]


def _final(outp, b2):
    blk = 2000
    return pl.pallas_call(
        _final_body,
        grid=(N // blk,),
        in_specs=[
            pl.BlockSpec((NC, blk, C), lambda i: (0, i, 0)),
            pl.BlockSpec((1, C), lambda i: (0, 0)),
        ],
        out_specs=pl.BlockSpec((blk, C), lambda i: (i, 0)),
        out_shape=jax.ShapeDtypeStruct((N, C), f32),
    )(outp, b2)


def kernel(nhots, hindices, vindices, Wl1, bl1, Wl2, bl2, weights1, weights2,
           bias1, bias2):
    s = hindices[:NT, 0]
    o = vindices[:NT, 1]
    w1flat = weights1.reshape(NSEG, EMB)

    p2d, t1, t2 = _prep(nhots, Wl1, bl1.reshape(1, RP), Wl2, bl2.reshape(1, RP))
    p = p2d.reshape(NT)
    t1f = t1.reshape(R * RP)
    t2f = t2.reshape(R * RP)

    colsum, rowsum = _sums(s, o, p, t1f, t2f)
    w1n, rrec = _norm(w1flat, colsum, rowsum)
    hp = _pass1(s, o, p, t1f, w1n)
    gflat = _gtable(hp, weights2, bias1.reshape(1, EMB)).reshape(NSEG, C)
    outp = _pass2(s, o, p, t2f, rrec.reshape(NSEG), gflat)
    return _final(outp, bias2.reshape(1, C))


# speedup vs baseline: 1.0279x; 1.0279x over previous
import functools

import jax
import jax.numpy as jnp
from jax import lax
from jax.experimental import pallas as pl
from jax.experimental.pallas import tpu as pltpu
from jax.experimental.pallas import tpu_sc as plsc

N = 10000
R = 16
NT = 160000
RP = 16
EMB = 32
C = 16
E = RP * NT
NSEG = N * RP
NC = 2
NS = 16
NW = NC * NS
EPT = E // NW
i32 = jnp.int32
f32 = jnp.float32


def _mesh():
    return plsc.VectorSubcoreMesh(
        core_axis_name="c", subcore_axis_name="s", num_cores=NC, num_subcores=NS
    )


def _zero_fill(ref, n):
    z = jnp.zeros((16,), f32)

    def body(i, _):
        ref[pl.ds(i * 16, 16)] = z
        return 0

    lax.fori_loop(0, n // 16, body, 0)


def _zero_fill2d(ref, nrows, ncols):
    z = jnp.zeros((16,), f32)
    cpr = ncols // 16

    def body(i, _):
        ref[i // cpr, pl.ds((i % cpr) * 16, 16)] = z
        return 0

    lax.fori_loop(0, nrows * cpr, body, 0)


def _wid_r_t0():
    cid = lax.axis_index("c")
    sid = lax.axis_index("s")
    wid = cid * NS + sid
    return cid, sid, wid // 2, (wid % 2) * EPT


def _prep_body(nhots_ref, wl1_ref, bl1_ref, wl2_ref, bl2_ref,
               p_ref, t1_ref, t2_ref):
    pid = pl.program_id(0)
    nh = nhots_ref[...]
    ridx = lax.broadcasted_iota(i32, nh.shape, 1).astype(f32)
    p_ref[...] = jnp.sum(nh * ridx, axis=1, keepdims=True).astype(i32)

    @pl.when(pid == 0)
    def _():
        for w_ref, b_ref, t_ref in ((wl1_ref, bl1_ref, t1_ref),
                                    (wl2_ref, bl2_ref, t2_ref)):
            z = w_ref[...] + b_ref[...]
            z = z - jnp.max(z, axis=1, keepdims=True)
            ez = jnp.exp(z)
            t_ref[...] = ez / jnp.sum(ez, axis=1, keepdims=True)


def _prep(nhots, wl1, bl1, wl2, bl2):
    blk = 4000
    return pl.pallas_call(
        _prep_body,
        grid=(NT // blk,),
        in_specs=[
            pl.BlockSpec((blk, R), lambda i: (i, 0)),
            pl.BlockSpec((R, RP), lambda i: (0, 0)),
            pl.BlockSpec((1, RP), lambda i: (0, 0)),
            pl.BlockSpec((R, RP), lambda i: (0, 0)),
            pl.BlockSpec((1, RP), lambda i: (0, 0)),
        ],
        out_specs=[
            pl.BlockSpec((blk, 1), lambda i: (i, 0)),
            pl.BlockSpec((R, RP), lambda i: (0, 0)),
            pl.BlockSpec((R, RP), lambda i: (0, 0)),
        ],
        out_shape=[
            jax.ShapeDtypeStruct((NT, 1), i32),
            jax.ShapeDtypeStruct((R, RP), f32),
            jax.ShapeDtypeStruct((R, RP), f32),
        ],
    )(nhots, wl1, bl1, wl2, bl2)


_AB = 3200


def _sums_body(s_hbm, o_hbm, p_hbm, t1_hbm, t2_hbm,
               colsum_hbm, rowsum_hbm,
               csum_sp, rsum_sp, t1v, t2v, sbuf, obuf, pbuf,
               colv, rowv, v1v, v2v, zb, sem, isem):
    cid, sid, r, t0 = _wid_r_t0()
    B = _AB
    KB = B // 128

    _zero_fill(zb, 2000)

    def zbody(i, _):
        pltpu.sync_copy(zb, csum_sp.at[pl.ds(sid * 10000 + i * 2000, 2000)])
        pltpu.sync_copy(zb, rsum_sp.at[pl.ds(sid * 10000 + i * 2000, 2000)])
        return 0

    lax.fori_loop(0, 5, zbody, 0)
    pltpu.sync_copy(t1_hbm, t1v)
    pltpu.sync_copy(t2_hbm, t2v)
    plsc.subcore_barrier()

    NBLK = EPT // B

    def fire_inputs(blk):
        st = blk & 1
        tb = t0 + blk * B
        pltpu.async_copy(s_hbm.at[pl.ds(tb, B)], sbuf.at[st], isem)
        pltpu.async_copy(o_hbm.at[pl.ds(tb, B)], obuf.at[st], isem)
        pltpu.async_copy(p_hbm.at[pl.ds(tb, B)], pbuf.at[st], isem)

    def wait_inputs(blk):
        st = blk & 1
        tb = t0 + blk * B
        pltpu.make_async_copy(s_hbm.at[pl.ds(tb, B)], sbuf.at[st], isem).wait()
        pltpu.make_async_copy(o_hbm.at[pl.ds(tb, B)], obuf.at[st], isem).wait()
        pltpu.make_async_copy(p_hbm.at[pl.ds(tb, B)], pbuf.at[st], isem).wait()

    def index_compute(st):
        @plsc.parallel_loop(0, B // 16, unroll=2)
        def inner(g):
            sv = sbuf[st, pl.ds(g * 16, 16)]
            ov = obuf[st, pl.ds(g * 16, 16)]
            pv = pbuf[st, pl.ds(g * 16, 16)]
            j, off = g // 8, (g % 8) * 16
            colv[st, j, pl.ds(off, 16)] = ov * r
            rowv[st, j, pl.ds(off, 16)] = sv * r
            tidx = pv * RP + r
            v1v[st, j, pl.ds(off, 16)] = plsc.load_gather(t1v, [tidx])
            v2v[st, j, pl.ds(off, 16)] = plsc.load_gather(t2v, [tidx])

    def fire_scatters(st):
        for j in range(KB):
            pltpu.async_copy(v1v.at[st, j], csum_sp.at[colv.at[st, j]], sem,
                             add=True)
            pltpu.async_copy(v2v.at[st, j], rsum_sp.at[rowv.at[st, j]], sem,
                             add=True)

    def wait_scatters(st):
        for j in range(KB):
            pltpu.make_async_copy(v1v.at[st, j], csum_sp.at[colv.at[st, j]],
                                  sem).wait()
            pltpu.make_async_copy(v2v.at[st, j], rsum_sp.at[rowv.at[st, j]],
                                  sem).wait()

    @pl.when(r > 0)
    def _():
        fire_inputs(0)
        wait_inputs(0)
        index_compute(0)
        fire_inputs(1)

        def block(blk, _):
            st = blk & 1
            nxt = 1 - st
            fire_scatters(st)

            @pl.when((blk > 0) & (blk < NBLK - 1))
            def _():
                wait_scatters(nxt)

            @pl.when(blk < NBLK - 2)
            def _():
                fire_inputs(blk + 2)

            @pl.when(blk < NBLK - 1)
            def _():
                wait_inputs(blk + 1)
                index_compute(nxt)

            return 0

        lax.fori_loop(0, NBLK, block, 0)
        wait_scatters((NBLK - 2) & 1)
        wait_scatters((NBLK - 1) & 1)

    @pl.when(r == 0)
    def _():
        def block(blk, acc):
            tb = t0 + blk * B
            pltpu.sync_copy(p_hbm.at[pl.ds(tb, B)], pbuf.at[0])

            def inner(g, acc2):
                a1, a2 = acc2
                pv = pbuf[0, pl.ds(g * 16, 16)]
                a1 = a1 + plsc.load_gather(t1v, [pv * RP])
                a2 = a2 + plsc.load_gather(t2v, [pv * RP])
                return a1, a2

            return lax.fori_loop(0, B // 16, inner, acc)

        z16 = jnp.zeros((16,), f32)
        zi16 = jnp.zeros((16,), i32)
        a1, a2 = lax.fori_loop(0, EPT // B, block, (z16, z16))
        for k in range(8):
            v1v[0, 0, pl.ds(k * 16, 16)] = z16
            v2v[0, 0, pl.ds(k * 16, 16)] = z16
            colv[0, 0, pl.ds(k * 16, 16)] = zi16
            rowv[0, 0, pl.ds(k * 16, 16)] = zi16
        v1v[0, 0, pl.ds(0, 16)] = a1
        v2v[0, 0, pl.ds(0, 16)] = a2
        pltpu.async_copy(v1v.at[0, 0], csum_sp.at[colv.at[0, 0]], sem,
                         add=True).wait()
        pltpu.async_copy(v2v.at[0, 0], rsum_sp.at[rowv.at[0, 0]], sem,
                         add=True).wait()

    plsc.subcore_barrier()

    @pl.when(sid == 0)
    def _():
        pltpu.sync_copy(csum_sp, colsum_hbm.at[cid])
        pltpu.sync_copy(rsum_sp, rowsum_hbm.at[cid])


def _sums(s, o, p, t1f, t2f):
    B = _AB
    return pl.kernel(
        _sums_body,
        out_type=[
            jax.ShapeDtypeStruct((NC, NSEG), f32),
            jax.ShapeDtypeStruct((NC, NSEG), f32),
        ],
        mesh=_mesh(),
        compiler_params=pltpu.CompilerParams(needs_layout_passes=False, use_tc_tiling_on_sc=False),
        scratch_types=[
            pltpu.VMEM_SHARED((NSEG,), f32),
            pltpu.VMEM_SHARED((NSEG,), f32),
            pltpu.VMEM((R * RP,), f32),
            pltpu.VMEM((R * RP,), f32),
            pltpu.VMEM((2, B), i32),
            pltpu.VMEM((2, B), i32),
            pltpu.VMEM((2, B), i32),
            pltpu.VMEM((2, B // 128, 128), i32),
            pltpu.VMEM((2, B // 128, 128), i32),
            pltpu.VMEM((2, B // 128, 128), f32),
            pltpu.VMEM((2, B // 128, 128), f32),
            pltpu.VMEM((2000,), f32),
            pltpu.SemaphoreType.DMA,
            pltpu.SemaphoreType.DMA,
        ],
    )(s, o, p, t1f, t2f)


def _norm_body(w1_ref, cs_ref, rs_ref, w1n_ref, rrec_ref):
    cs = cs_ref[0] + cs_ref[1]
    rs = rs_ref[0] + rs_ref[1]
    w1n_ref[...] = w1_ref[...] / cs
    rrec_ref[...] = 1.0 / rs


def _norm(w1flat, colsum, rowsum):
    blk = 3200
    return pl.pallas_call(
        _norm_body,
        grid=(NSEG // blk,),
        in_specs=[
            pl.BlockSpec((blk, EMB), lambda i: (i, 0)),
            pl.BlockSpec((NC, blk, 1), lambda i: (0, i, 0)),
            pl.BlockSpec((NC, blk, 1), lambda i: (0, i, 0)),
        ],
        out_specs=[
            pl.BlockSpec((blk, EMB), lambda i: (i, 0)),
            pl.BlockSpec((blk, 1), lambda i: (i, 0)),
        ],
        out_shape=[
            jax.ShapeDtypeStruct((NSEG, EMB), f32),
            jax.ShapeDtypeStruct((NSEG, 1), f32),
        ],
    )(w1flat, colsum.reshape(NC, NSEG, 1), rowsum.reshape(NC, NSEG, 1))


_CB = 640


def _pass1_body(s_hbm, o_hbm, p_hbm, t1_hbm, w1n_hbm, hp_hbm,
                h_sp, t1v, sbuf, obuf, pbuf, gidxv, dstv, facv, rows,
                zrows, w1r0, isem, gsem, ssem):
    cid, sid, r, t0 = _wid_r_t0()
    B = _CB
    KB = B // 128

    _zero_fill2d(zrows, 125, EMB)

    def zbody(i, _):
        pltpu.sync_copy(zrows, h_sp.at[pl.ds(sid * 625 + i * 125, 125), :])
        return 0

    lax.fori_loop(0, 5, zbody, 0)
    pltpu.sync_copy(t1_hbm, t1v)
    pltpu.sync_copy(w1n_hbm.at[0], w1r0)
    w0a = w1r0[pl.ds(0, 16)]
    w0b = w1r0[pl.ds(16, 16)]
    plsc.subcore_barrier()

    NBLK = EPT // B

    def fire_inputs(blk):
        st = blk & 1
        tb = t0 + blk * B
        pltpu.async_copy(s_hbm.at[pl.ds(tb, B)], sbuf.at[st], isem)
        pltpu.async_copy(o_hbm.at[pl.ds(tb, B)], obuf.at[st], isem)
        pltpu.async_copy(p_hbm.at[pl.ds(tb, B)], pbuf.at[st], isem)

    def wait_inputs(blk):
        st = blk & 1
        tb = t0 + blk * B
        pltpu.make_async_copy(s_hbm.at[pl.ds(tb, B)], sbuf.at[st], isem).wait()
        pltpu.make_async_copy(o_hbm.at[pl.ds(tb, B)], obuf.at[st], isem).wait()
        pltpu.make_async_copy(p_hbm.at[pl.ds(tb, B)], pbuf.at[st], isem).wait()

    def index_compute(st):
        @plsc.parallel_loop(0, B // 16, unroll=2)
        def inner(g):
            sv = sbuf[st, pl.ds(g * 16, 16)]
            ov = obuf[st, pl.ds(g * 16, 16)]
            pv = pbuf[st, pl.ds(g * 16, 16)]
            gidxv[st, pl.ds(g * 16, 16)] = ov * r
            j, off = g // 8, (g % 8) * 16
            dstv[st, j, pl.ds(off, 16)] = sv
            facv[st, pl.ds(g * 16, 16)] = plsc.load_gather(t1v, [pv * RP + r])

    def fire_gather(st):
        @pl.when(r > 0)
        def _():
            pltpu.async_copy(w1n_hbm.at[gidxv.at[st]], rows.at[st], gsem)

    def wait_gather(st):
        @pl.when(r > 0)
        def _():
            pltpu.make_async_copy(w1n_hbm.at[gidxv.at[st]], rows.at[st],
                                  gsem).wait()

    def scale(st):
        @pl.when(r > 0)
        def _():
            @plsc.parallel_loop(0, B // 16, unroll=2)
            def body(g):
                for b2 in range(16):
                    e = g * 16 + b2
                    fv = plsc.load_gather(facv.at[st],
                                          [jnp.full((16,), 0, i32) + e])
                    rows[st, e, pl.ds(0, 16)] = rows[st, e, pl.ds(0, 16)] * fv
                    rows[st, e, pl.ds(16, 16)] = rows[st, e, pl.ds(16, 16)] * fv

        @pl.when(r == 0)
        def _():
            @plsc.parallel_loop(0, B // 16, unroll=2)
            def body0(g):
                for b2 in range(16):
                    e = g * 16 + b2
                    fv = plsc.load_gather(facv.at[st],
                                          [jnp.full((16,), 0, i32) + e])
                    rows[st, e, pl.ds(0, 16)] = w0a * fv
                    rows[st, e, pl.ds(16, 16)] = w0b * fv

    def fire_scatters(st):
        for j in range(KB):
            pltpu.async_copy(rows.at[st, pl.ds(j * 128, 128), :],
                             h_sp.at[dstv.at[st, j]], ssem, add=True)

    def wait_scatters(st):
        for j in range(KB):
            pltpu.make_async_copy(rows.at[st, pl.ds(j * 128, 128), :],
                                  h_sp.at[dstv.at[st, j]], ssem).wait()

    fire_inputs(0)
    wait_inputs(0)
    index_compute(0)
    fire_gather(0)
    fire_inputs(1)

    def block(blk, _):
        st = blk & 1
        nxt = 1 - st
        wait_gather(st)
        scale(st)
        fire_scatters(st)

        @pl.when((blk > 0) & (blk < NBLK - 1))
        def _():
            wait_scatters(nxt)

        @pl.when(blk < NBLK - 2)
        def _():
            fire_inputs(blk + 2)

        @pl.when(blk < NBLK - 1)
        def _():
            wait_inputs(blk + 1)
            index_compute(nxt)
            fire_gather(nxt)

        return 0

    lax.fori_loop(0, NBLK, block, 0)
    wait_scatters((NBLK - 2) & 1)
    wait_scatters((NBLK - 1) & 1)
    plsc.subcore_barrier()

    @pl.when(sid == 0)
    def _():
        pltpu.sync_copy(h_sp, hp_hbm.at[cid])


def _pass1(s, o, p, t1f, w1n):
    B = _CB
    return pl.kernel(
        _pass1_body,
        out_type=jax.ShapeDtypeStruct((NC, N, EMB), f32),
        mesh=_mesh(),
        compiler_params=pltpu.CompilerParams(needs_layout_passes=False, use_tc_tiling_on_sc=False),
        scratch_types=[
            pltpu.VMEM_SHARED((N, EMB), f32),
            pltpu.VMEM((R * RP,), f32),
            pltpu.VMEM((2, B), i32),
            pltpu.VMEM((2, B), i32),
            pltpu.VMEM((2, B), i32),
            pltpu.VMEM((2, B), i32),
            pltpu.VMEM((2, B // 128, 128), i32),
            pltpu.VMEM((2, B), f32),
            pltpu.VMEM((2, B, EMB), f32),
            pltpu.VMEM((125, EMB), f32),
            pltpu.VMEM((EMB,), f32),
            pltpu.SemaphoreType.DMA,
            pltpu.SemaphoreType.DMA,
            pltpu.SemaphoreType.DMA,
        ],
    )(s, o, p, t1f, w1n)


def _g_body(hp_ref, w2_ref, b1_ref, g_ref):
    h = jnp.maximum(hp_ref[0] + hp_ref[1] + b1_ref[...], 0.0)
    g_ref[0] = jnp.dot(h, w2_ref[0], preferred_element_type=f32)


def _gtable(hp, w2, b1):
    blk = 2000
    return pl.pallas_call(
        _g_body,
        grid=(RP, N // blk),
        in_specs=[
            pl.BlockSpec((NC, blk, EMB), lambda r, i: (0, i, 0)),
            pl.BlockSpec((1, EMB, C), lambda r, i: (r, 0, 0)),
            pl.BlockSpec((1, EMB), lambda r, i: (0, 0)),
        ],
        out_specs=pl.BlockSpec((1, blk, C), lambda r, i: (r, i, 0)),
        out_shape=jax.ShapeDtypeStruct((RP, N, C), f32),
    )(hp, w2, b1)


def _pass2_body(s_hbm, o_hbm, p_hbm, t2_hbm, rrec_hbm, g_hbm, outp_hbm,
                out_sp, t2v, rrv, rstage, sbuf, obuf, pbuf, gidxv, dstv, facv,
                rows, zrows, isem, gsem, ssem):
    cid, sid, r, t0 = _wid_r_t0()
    B = _CB
    KB = B // 128

    _zero_fill2d(zrows, 125, C)

    def zbody(i, _):
        pltpu.sync_copy(zrows, out_sp.at[pl.ds(sid * 625 + i * 125, 125), :])
        return 0

    lax.fori_loop(0, 5, zbody, 0)
    pltpu.sync_copy(t2_hbm, t2v)

    lane = lax.iota(i32, 16)
    for q in range(5):
        pltpu.sync_copy(rrec_hbm.at[pl.ds(q * 2000 * r, 32000)], rstage)

        @plsc.parallel_loop(0, 125)
        def rbody(g):
            idx = (g * 16 + lane) * r
            rrv[pl.ds(q * 2000 + g * 16, 16)] = plsc.load_gather(rstage, [idx])

    plsc.subcore_barrier()

    NBLK = EPT // B

    def fire_inputs(blk):
        st = blk & 1
        tb = t0 + blk * B
        pltpu.async_copy(s_hbm.at[pl.ds(tb, B)], sbuf.at[st], isem)
        pltpu.async_copy(o_hbm.at[pl.ds(tb, B)], obuf.at[st], isem)
        pltpu.async_copy(p_hbm.at[pl.ds(tb, B)], pbuf.at[st], isem)

    def wait_inputs(blk):
        st = blk & 1
        tb = t0 + blk * B
        pltpu.make_async_copy(s_hbm.at[pl.ds(tb, B)], sbuf.at[st], isem).wait()
        pltpu.make_async_copy(o_hbm.at[pl.ds(tb, B)], obuf.at[st], isem).wait()
        pltpu.make_async_copy(p_hbm.at[pl.ds(tb, B)], pbuf.at[st], isem).wait()

    def index_compute(st):
        @plsc.parallel_loop(0, B // 16, unroll=2)
        def inner(g):
            sv = sbuf[st, pl.ds(g * 16, 16)]
            ov = obuf[st, pl.ds(g * 16, 16)]
            pv = pbuf[st, pl.ds(g * 16, 16)]
            rowi = sv * r
            rpv = ((rowi.astype(f32) + 0.5) * (1.0 / N)).astype(i32)
            gidxv[st, pl.ds(g * 16, 16)] = rpv * N + ov
            j, off = g // 8, (g % 8) * 16
            dstv[st, j, pl.ds(off, 16)] = rowi - rpv * N
            t2g = plsc.load_gather(t2v, [pv * RP + r])
            rrg = plsc.load_gather(rrv, [sv])
            facv[st, pl.ds(g * 16, 16)] = t2g * rrg

    def fire_gather(st):
        pltpu.async_copy(g_hbm.at[gidxv.at[st]], rows.at[st], gsem)

    def wait_gather(st):
        pltpu.make_async_copy(g_hbm.at[gidxv.at[st]], rows.at[st], gsem).wait()

    def scale(st):
        @plsc.parallel_loop(0, B // 16, unroll=2)
        def body(g):
            for b2 in range(16):
                e = g * 16 + b2
                fv = plsc.load_gather(facv.at[st],
                                      [jnp.full((16,), 0, i32) + e])
                rows[st, e, pl.ds(0, 16)] = rows[st, e, pl.ds(0, 16)] * fv

    def fire_scatters(st):
        for j in range(KB):
            pltpu.async_copy(rows.at[st, pl.ds(j * 128, 128), :],
                             out_sp.at[dstv.at[st, j]], ssem, add=True)

    def wait_scatters(st):
        for j in range(KB):
            pltpu.make_async_copy(rows.at[st, pl.ds(j * 128, 128), :],
                                  out_sp.at[dstv.at[st, j]], ssem).wait()

    fire_inputs(0)
    wait_inputs(0)
    index_compute(0)
    fire_gather(0)
    fire_inputs(1)

    def block(blk, _):
        st = blk & 1
        nxt = 1 - st
        wait_gather(st)
        scale(st)
        fire_scatters(st)

        @pl.when((blk > 0) & (blk < NBLK - 1))
        def _():
            wait_scatters(nxt)

        @pl.when(blk < NBLK - 2)
        def _():
            fire_inputs(blk + 2)

        @pl.when(blk < NBLK - 1)
        def _():
            wait_inputs(blk + 1)
            index_compute(nxt)
            fire_gather(nxt)

        return 0

    lax.fori_loop(0, NBLK, block, 0)
    wait_scatters((NBLK - 2) & 1)
    wait_scatters((NBLK - 1) & 1)
    plsc.subcore_barrier()

    @pl.when(sid == 0)
    def _():
        pltpu.sync_copy(out_sp, outp_hbm.at[cid])


def _pass2(s, o, p, t2f, rrec, gflat):
    B = _CB
    return pl.kernel(
        _pass2_body,
        out_type=jax.ShapeDtypeStruct((NC, N, C), f32),
        mesh=_mesh(),
        compiler_params=pltpu.CompilerParams(needs_layout_passes=False, use_tc_tiling_on_sc=False),
        scratch_types=[
            pltpu.VMEM_SHARED((N, C), f32),
            pltpu.VMEM((R * RP,), f32),
            pltpu.VMEM((N,), f32),
            pltpu.VMEM((32000,), f32),
            pltpu.VMEM((2, B), i32),
            pltpu.VMEM((2, B), i32),
            pltpu.VMEM((2, B), i32),
            pltpu.VMEM((2, B), i32),
            pltpu.VMEM((2, B // 128, 128), i32),
            pltpu.VMEM((2, B), f32),
            pltpu.VMEM((2, B, C), f32),
            pltpu.VMEM((125, C), f32),
            pltpu.SemaphoreType.DMA,
            pltpu.SemaphoreType.DMA,
            pltpu.SemaphoreType.DMA,
        ],
    )(s, o, p, t2f, rrec, gflat)


def _final_body(op_ref, b2_ref, out_ref):
    out_ref[...] = op_ref[0] + op_ref[1] + b2_ref[...]


def _final(outp, b2):
    blk = 2000
    return pl.pallas_call(
        _final_body,
        grid=(N // blk,),
        in_specs=[
            pl.BlockSpec((NC, blk, C), lambda i: (0, i, 0)),
            pl.BlockSpec((1, C), lambda i: (0, 0)),
        ],
        out_specs=pl.BlockSpec((blk, C), lambda i: (i, 0)),
        out_shape=jax.ShapeDtypeStruct((N, C), f32),
    )(outp, b2)


def kernel(nhots, hindices, vindices, Wl1, bl1, Wl2, bl2, weights1, weights2,
           bias1, bias2):
    s = hindices[:NT, 0]
    o = vindices[:NT, 1]
    w1flat = weights1.reshape(NSEG, EMB)

    p2d, t1, t2 = _prep(nhots, Wl1, bl1.reshape(1, RP), Wl2, bl2.reshape(1, RP))
    p = p2d.reshape(NT)
    t1f = t1.reshape(R * RP)
    t2f = t2.reshape(R * RP)

    colsum, rowsum = _sums(s, o, p, t1f, t2f)
    w1n, rrec = _norm(w1flat, colsum, rowsum)
    hp = _pass1(s, o, p, t1f, w1n)
    gflat = _gtable(hp, weights2, bias1.reshape(1, EMB)).reshape(NSEG, C)
    outp = _pass2(s, o, p, t2f, rrec.reshape(NSEG), gflat)
    return _final(outp, bias2.reshape(1, C))


# speedup vs baseline: 1.0289x; 1.0010x over previous
import jax
import jax.numpy as jnp
from jax import lax
from jax.experimental import pallas as pl
from jax.experimental.pallas import tpu as pltpu
from jax.experimental.pallas import tpu_sc as plsc

N = 10000
R = 16
NT = 160000
RP = 16
EMB = 32
C = 16
E = RP * NT
NSEG = N * RP
NC = 2
NS = 16
NW = NC * NS
EPT = E // NW
i32 = jnp.int32
f32 = jnp.float32


def _mesh():
    return plsc.VectorSubcoreMesh(
        core_axis_name="c", subcore_axis_name="s", num_cores=NC, num_subcores=NS
    )


def _zero_fill(ref, n):
    z = jnp.zeros((16,), f32)

    def body(i, _):
        ref[pl.ds(i * 16, 16)] = z
        return 0

    lax.fori_loop(0, n // 16, body, 0)


def _zero_fill2d(ref, nrows, ncols):
    z = jnp.zeros((16,), f32)
    cpr = ncols // 16

    def body(i, _):
        ref[i // cpr, pl.ds((i % cpr) * 16, 16)] = z
        return 0

    lax.fori_loop(0, nrows * cpr, body, 0)


def _wid_r_t0():
    cid = lax.axis_index("c")
    sid = lax.axis_index("s")
    wid = cid * NS + sid
    return cid, sid, wid // 2, (wid % 2) * EPT


def _prep_body(nhots_ref, wl1_ref, bl1_ref, wl2_ref, bl2_ref,
               p_ref, t1_ref, t2_ref):
    pid = pl.program_id(0)
    nh = nhots_ref[...]
    ridx = lax.broadcasted_iota(i32, nh.shape, 1).astype(f32)
    p_ref[...] = jnp.sum(nh * ridx, axis=1, keepdims=True).astype(i32)

    @pl.when(pid == 0)
    def _():
        for w_ref, b_ref, t_ref in ((wl1_ref, bl1_ref, t1_ref),
                                    (wl2_ref, bl2_ref, t2_ref)):
            z = w_ref[...] + b_ref[...]
            z = z - jnp.max(z, axis=1, keepdims=True)
            ez = jnp.exp(z)
            t_ref[...] = ez / jnp.sum(ez, axis=1, keepdims=True)


def _prep(nhots, wl1, bl1, wl2, bl2):
    blk = 4000
    return pl.pallas_call(
        _prep_body,
        grid=(NT // blk,),
        in_specs=[
            pl.BlockSpec((blk, R), lambda i: (i, 0)),
            pl.BlockSpec((R, RP), lambda i: (0, 0)),
            pl.BlockSpec((1, RP), lambda i: (0, 0)),
            pl.BlockSpec((R, RP), lambda i: (0, 0)),
            pl.BlockSpec((1, RP), lambda i: (0, 0)),
        ],
        out_specs=[
            pl.BlockSpec((blk, 1), lambda i: (i, 0)),
            pl.BlockSpec((R, RP), lambda i: (0, 0)),
            pl.BlockSpec((R, RP), lambda i: (0, 0)),
        ],
        out_shape=[
            jax.ShapeDtypeStruct((NT, 1), i32),
            jax.ShapeDtypeStruct((R, RP), f32),
            jax.ShapeDtypeStruct((R, RP), f32),
        ],
    )(nhots, wl1, bl1, wl2, bl2)


_AB = 3200


def _sums_body(s_hbm, o_hbm, p_hbm, t1_hbm, t2_hbm,
               colsum_hbm, rowsum_hbm,
               csum_sp, rsum_sp, t1v, t2v, sbuf, obuf, pbuf,
               colv, rowv, v1v, v2v, zb, sem, isem):
    cid, sid, r, t0 = _wid_r_t0()
    B = _AB
    KB = B // 128

    _zero_fill(zb, 2000)

    def zbody(i, _):
        pltpu.sync_copy(zb, csum_sp.at[pl.ds(sid * 10000 + i * 2000, 2000)])
        pltpu.sync_copy(zb, rsum_sp.at[pl.ds(sid * 10000 + i * 2000, 2000)])
        return 0

    lax.fori_loop(0, 5, zbody, 0)
    pltpu.sync_copy(t1_hbm, t1v)
    pltpu.sync_copy(t2_hbm, t2v)
    plsc.subcore_barrier()

    NBLK = EPT // B

    def fire_inputs(blk):
        st = blk & 1
        tb = t0 + blk * B
        pltpu.async_copy(s_hbm.at[pl.ds(tb, B)], sbuf.at[st], isem)
        pltpu.async_copy(o_hbm.at[pl.ds(tb, B)], obuf.at[st], isem)
        pltpu.async_copy(p_hbm.at[pl.ds(tb, B)], pbuf.at[st], isem)

    def wait_inputs(blk):
        st = blk & 1
        tb = t0 + blk * B
        pltpu.make_async_copy(s_hbm.at[pl.ds(tb, B)], sbuf.at[st], isem).wait()
        pltpu.make_async_copy(o_hbm.at[pl.ds(tb, B)], obuf.at[st], isem).wait()
        pltpu.make_async_copy(p_hbm.at[pl.ds(tb, B)], pbuf.at[st], isem).wait()

    def index_compute(st):
        @plsc.parallel_loop(0, B // 16, unroll=2)
        def inner(g):
            sv = sbuf[st, pl.ds(g * 16, 16)]
            ov = obuf[st, pl.ds(g * 16, 16)]
            pv = pbuf[st, pl.ds(g * 16, 16)]
            j, off = g // 8, (g % 8) * 16
            colv[st, j, pl.ds(off, 16)] = ov * r
            rowv[st, j, pl.ds(off, 16)] = sv * r
            tidx = pv * RP + r
            v1v[st, j, pl.ds(off, 16)] = plsc.load_gather(t1v, [tidx])
            v2v[st, j, pl.ds(off, 16)] = plsc.load_gather(t2v, [tidx])

    def fire_scatters(st):
        for j in range(KB):
            pltpu.async_copy(v1v.at[st, j], csum_sp.at[colv.at[st, j]], sem,
                             add=True)
            pltpu.async_copy(v2v.at[st, j], rsum_sp.at[rowv.at[st, j]], sem,
                             add=True)

    def wait_scatters(st):
        for j in range(KB):
            pltpu.make_async_copy(v1v.at[st, j], csum_sp.at[colv.at[st, j]],
                                  sem).wait()
            pltpu.make_async_copy(v2v.at[st, j], rsum_sp.at[rowv.at[st, j]],
                                  sem).wait()

    @pl.when(r > 0)
    def _():
        fire_inputs(0)
        wait_inputs(0)
        index_compute(0)
        fire_inputs(1)

        def block(blk, _):
            st = blk & 1
            nxt = 1 - st
            fire_scatters(st)

            @pl.when((blk > 0) & (blk < NBLK - 1))
            def _():
                wait_scatters(nxt)

            @pl.when(blk < NBLK - 2)
            def _():
                fire_inputs(blk + 2)

            @pl.when(blk < NBLK - 1)
            def _():
                wait_inputs(blk + 1)
                index_compute(nxt)

            return 0

        lax.fori_loop(0, NBLK, block, 0)
        wait_scatters((NBLK - 2) & 1)
        wait_scatters((NBLK - 1) & 1)

    @pl.when(r == 0)
    def _():
        def block(blk, acc):
            tb = t0 + blk * B
            pltpu.sync_copy(p_hbm.at[pl.ds(tb, B)], pbuf.at[0])

            def inner(g, acc2):
                a1, a2 = acc2
                pv = pbuf[0, pl.ds(g * 16, 16)]
                a1 = a1 + plsc.load_gather(t1v, [pv * RP])
                a2 = a2 + plsc.load_gather(t2v, [pv * RP])
                return a1, a2

            return lax.fori_loop(0, B // 16, inner, acc)

        z16 = jnp.zeros((16,), f32)
        zi16 = jnp.zeros((16,), i32)
        a1, a2 = lax.fori_loop(0, EPT // B, block, (z16, z16))
        for k in range(8):
            v1v[0, 0, pl.ds(k * 16, 16)] = z16
            v2v[0, 0, pl.ds(k * 16, 16)] = z16
            colv[0, 0, pl.ds(k * 16, 16)] = zi16
            rowv[0, 0, pl.ds(k * 16, 16)] = zi16
        v1v[0, 0, pl.ds(0, 16)] = a1
        v2v[0, 0, pl.ds(0, 16)] = a2
        pltpu.async_copy(v1v.at[0, 0], csum_sp.at[colv.at[0, 0]], sem,
                         add=True).wait()
        pltpu.async_copy(v2v.at[0, 0], rsum_sp.at[rowv.at[0, 0]], sem,
                         add=True).wait()

    plsc.subcore_barrier()

    @pl.when(sid == 0)
    def _():
        pltpu.sync_copy(csum_sp, colsum_hbm.at[cid])
        pltpu.sync_copy(rsum_sp, rowsum_hbm.at[cid])


def _sums(s, o, p, t1f, t2f):
    B = _AB
    return pl.kernel(
        _sums_body,
        out_type=[
            jax.ShapeDtypeStruct((NC, NSEG), f32),
            jax.ShapeDtypeStruct((NC, NSEG), f32),
        ],
        mesh=_mesh(),
        compiler_params=pltpu.CompilerParams(needs_layout_passes=False, use_tc_tiling_on_sc=False),
        scratch_types=[
            pltpu.VMEM_SHARED((NSEG,), f32),
            pltpu.VMEM_SHARED((NSEG,), f32),
            pltpu.VMEM((R * RP,), f32),
            pltpu.VMEM((R * RP,), f32),
            pltpu.VMEM((2, B), i32),
            pltpu.VMEM((2, B), i32),
            pltpu.VMEM((2, B), i32),
            pltpu.VMEM((2, B // 128, 128), i32),
            pltpu.VMEM((2, B // 128, 128), i32),
            pltpu.VMEM((2, B // 128, 128), f32),
            pltpu.VMEM((2, B // 128, 128), f32),
            pltpu.VMEM((2000,), f32),
            pltpu.SemaphoreType.DMA,
            pltpu.SemaphoreType.DMA,
        ],
    )(s, o, p, t1f, t2f)


def _norm_body(w1_ref, cs_ref, rs_ref, w1n_ref, rrec_ref):
    cs = cs_ref[0] + cs_ref[1]
    rs = rs_ref[0] + rs_ref[1]
    w1n_ref[...] = w1_ref[...] / cs
    rrec_ref[...] = 1.0 / rs


def _norm(w1flat, colsum, rowsum):
    blk = 3200
    return pl.pallas_call(
        _norm_body,
        grid=(NSEG // blk,),
        in_specs=[
            pl.BlockSpec((blk, EMB), lambda i: (i, 0)),
            pl.BlockSpec((NC, blk, 1), lambda i: (0, i, 0)),
            pl.BlockSpec((NC, blk, 1), lambda i: (0, i, 0)),
        ],
        out_specs=[
            pl.BlockSpec((blk, EMB), lambda i: (i, 0)),
            pl.BlockSpec((blk, 1), lambda i: (i, 0)),
        ],
        out_shape=[
            jax.ShapeDtypeStruct((NSEG, EMB), f32),
            jax.ShapeDtypeStruct((NSEG, 1), f32),
        ],
    )(w1flat, colsum.reshape(NC, NSEG, 1), rowsum.reshape(NC, NSEG, 1))


_CB = 640


def _pass1_body(s_hbm, o_hbm, p_hbm, t1_hbm, w1n_hbm, hp_hbm,
                h_sp, t1v, sbuf, obuf, pbuf, gidxv, dstv, facv, rows,
                zrows, w1r0, isem, gsem, ssem):
    cid, sid, r, t0 = _wid_r_t0()
    B = _CB
    KB = B // 128

    _zero_fill2d(zrows, 125, EMB)

    def zbody(i, _):
        pltpu.sync_copy(zrows, h_sp.at[pl.ds(sid * 625 + i * 125, 125), :])
        return 0

    lax.fori_loop(0, 5, zbody, 0)
    pltpu.sync_copy(t1_hbm, t1v)
    pltpu.sync_copy(w1n_hbm.at[0], w1r0)
    w0a = w1r0[pl.ds(0, 16)]
    w0b = w1r0[pl.ds(16, 16)]
    plsc.subcore_barrier()

    NBLK = EPT // B

    def fire_inputs(blk):
        st = blk & 1
        tb = t0 + blk * B
        pltpu.async_copy(s_hbm.at[pl.ds(tb, B)], sbuf.at[st], isem)
        pltpu.async_copy(o_hbm.at[pl.ds(tb, B)], obuf.at[st], isem)
        pltpu.async_copy(p_hbm.at[pl.ds(tb, B)], pbuf.at[st], isem)

    def wait_inputs(blk):
        st = blk & 1
        tb = t0 + blk * B
        pltpu.make_async_copy(s_hbm.at[pl.ds(tb, B)], sbuf.at[st], isem).wait()
        pltpu.make_async_copy(o_hbm.at[pl.ds(tb, B)], obuf.at[st], isem).wait()
        pltpu.make_async_copy(p_hbm.at[pl.ds(tb, B)], pbuf.at[st], isem).wait()

    def index_compute(st):
        @plsc.parallel_loop(0, B // 16, unroll=2)
        def inner(g):
            sv = sbuf[st, pl.ds(g * 16, 16)]
            ov = obuf[st, pl.ds(g * 16, 16)]
            pv = pbuf[st, pl.ds(g * 16, 16)]
            gidxv[st, pl.ds(g * 16, 16)] = ov * r
            j, off = g // 8, (g % 8) * 16
            dstv[st, j, pl.ds(off, 16)] = sv
            facv[st, pl.ds(g * 16, 16)] = plsc.load_gather(t1v, [pv * RP + r])

    def fire_gather(st):
        @pl.when(r > 0)
        def _():
            pltpu.async_copy(w1n_hbm.at[gidxv.at[st]], rows.at[st], gsem)

    def wait_gather(st):
        @pl.when(r > 0)
        def _():
            pltpu.make_async_copy(w1n_hbm.at[gidxv.at[st]], rows.at[st],
                                  gsem).wait()

    def scale(st):
        @pl.when(r > 0)
        def _():
            @plsc.parallel_loop(0, B // 16, unroll=2)
            def body(g):
                for b2 in range(16):
                    e = g * 16 + b2
                    fv = plsc.load_gather(facv.at[st],
                                          [jnp.full((16,), 0, i32) + e])
                    rows[st, e, pl.ds(0, 16)] = rows[st, e, pl.ds(0, 16)] * fv
                    rows[st, e, pl.ds(16, 16)] = rows[st, e, pl.ds(16, 16)] * fv

        @pl.when(r == 0)
        def _():
            @plsc.parallel_loop(0, B // 16, unroll=2)
            def body0(g):
                for b2 in range(16):
                    e = g * 16 + b2
                    fv = plsc.load_gather(facv.at[st],
                                          [jnp.full((16,), 0, i32) + e])
                    rows[st, e, pl.ds(0, 16)] = w0a * fv
                    rows[st, e, pl.ds(16, 16)] = w0b * fv

    def fire_scatters(st):
        for j in range(KB):
            pltpu.async_copy(rows.at[st, pl.ds(j * 128, 128), :],
                             h_sp.at[dstv.at[st, j]], ssem, add=True)

    def wait_scatters(st):
        for j in range(KB):
            pltpu.make_async_copy(rows.at[st, pl.ds(j * 128, 128), :],
                                  h_sp.at[dstv.at[st, j]], ssem).wait()

    fire_inputs(0)
    wait_inputs(0)
    index_compute(0)
    fire_gather(0)
    fire_inputs(1)

    def block(blk, _):
        st = blk & 1
        nxt = 1 - st
        wait_gather(st)
        scale(st)
        fire_scatters(st)

        @pl.when((blk > 0) & (blk < NBLK - 1))
        def _():
            wait_scatters(nxt)

        @pl.when(blk < NBLK - 2)
        def _():
            fire_inputs(blk + 2)

        @pl.when(blk < NBLK - 1)
        def _():
            wait_inputs(blk + 1)
            index_compute(nxt)
            fire_gather(nxt)

        return 0

    lax.fori_loop(0, NBLK, block, 0)
    wait_scatters((NBLK - 2) & 1)
    wait_scatters((NBLK - 1) & 1)
    plsc.subcore_barrier()

    @pl.when(sid == 0)
    def _():
        pltpu.sync_copy(h_sp, hp_hbm.at[cid])


def _pass1(s, o, p, t1f, w1n):
    B = _CB
    return pl.kernel(
        _pass1_body,
        out_type=jax.ShapeDtypeStruct((NC, N, EMB), f32),
        mesh=_mesh(),
        compiler_params=pltpu.CompilerParams(needs_layout_passes=False, use_tc_tiling_on_sc=False),
        scratch_types=[
            pltpu.VMEM_SHARED((N, EMB), f32),
            pltpu.VMEM((R * RP,), f32),
            pltpu.VMEM((2, B), i32),
            pltpu.VMEM((2, B), i32),
            pltpu.VMEM((2, B), i32),
            pltpu.VMEM((2, B), i32),
            pltpu.VMEM((2, B // 128, 128), i32),
            pltpu.VMEM((2, B), f32),
            pltpu.VMEM((2, B, EMB), f32),
            pltpu.VMEM((125, EMB), f32),
            pltpu.VMEM((EMB,), f32),
            pltpu.SemaphoreType.DMA,
            pltpu.SemaphoreType.DMA,
            pltpu.SemaphoreType.DMA,
        ],
    )(s, o, p, t1f, w1n)


def _g_body(hp_ref, w2_ref, b1_ref, g_ref):
    h = jnp.maximum(hp_ref[0] + hp_ref[1] + b1_ref[...], 0.0)
    g_ref[0] = jnp.dot(h, w2_ref[0], preferred_element_type=f32)


def _gtable(hp, w2, b1):
    blk = 2000
    return pl.pallas_call(
        _g_body,
        grid=(RP, N // blk),
        in_specs=[
            pl.BlockSpec((NC, blk, EMB), lambda r, i: (0, i, 0)),
            pl.BlockSpec((1, EMB, C), lambda r, i: (r, 0, 0)),
            pl.BlockSpec((1, EMB), lambda r, i: (0, 0)),
        ],
        out_specs=pl.BlockSpec((1, blk, C), lambda r, i: (r, i, 0)),
        out_shape=jax.ShapeDtypeStruct((RP, N, C), f32),
    )(hp, w2, b1)


def _pass2_body(s_hbm, o_hbm, p_hbm, t2_hbm, rrec_hbm, g_hbm, outp_hbm,
                out_sp, t2v, rrv, rstage, sbuf, obuf, pbuf, gidxv, dstv, facv,
                rows, zrows, isem, gsem, ssem):
    cid, sid, r, t0 = _wid_r_t0()
    B = _CB
    KB = B // 128

    _zero_fill2d(zrows, 125, C)

    def zbody(i, _):
        pltpu.sync_copy(zrows, out_sp.at[pl.ds(sid * 625 + i * 125, 125), :])
        return 0

    lax.fori_loop(0, 5, zbody, 0)
    pltpu.sync_copy(t2_hbm, t2v)

    lane = lax.iota(i32, 16)
    for q in range(5):
        pltpu.sync_copy(rrec_hbm.at[pl.ds(q * 2000 * r, 32000)], rstage)

        @plsc.parallel_loop(0, 125)
        def rbody(g):
            idx = (g * 16 + lane) * r
            rrv[pl.ds(q * 2000 + g * 16, 16)] = plsc.load_gather(rstage, [idx])

    plsc.subcore_barrier()

    NBLK = EPT // B

    def fire_inputs(blk):
        st = blk & 1
        tb = t0 + blk * B
        pltpu.async_copy(s_hbm.at[pl.ds(tb, B)], sbuf.at[st], isem)
        pltpu.async_copy(o_hbm.at[pl.ds(tb, B)], obuf.at[st], isem)
        pltpu.async_copy(p_hbm.at[pl.ds(tb, B)], pbuf.at[st], isem)

    def wait_inputs(blk):
        st = blk & 1
        tb = t0 + blk * B
        pltpu.make_async_copy(s_hbm.at[pl.ds(tb, B)], sbuf.at[st], isem).wait()
        pltpu.make_async_copy(o_hbm.at[pl.ds(tb, B)], obuf.at[st], isem).wait()
        pltpu.make_async_copy(p_hbm.at[pl.ds(tb, B)], pbuf.at[st], isem).wait()

    def index_compute(st):
        @plsc.parallel_loop(0, B // 16, unroll=2)
        def inner(g):
            sv = sbuf[st, pl.ds(g * 16, 16)]
            ov = obuf[st, pl.ds(g * 16, 16)]
            pv = pbuf[st, pl.ds(g * 16, 16)]
            rowi = sv * r
            rpv = ((rowi.astype(f32) + 0.5) * (1.0 / N)).astype(i32)
            gidxv[st, pl.ds(g * 16, 16)] = rpv * N + ov
            j, off = g // 8, (g % 8) * 16
            dstv[st, j, pl.ds(off, 16)] = rowi - rpv * N
            t2g = plsc.load_gather(t2v, [pv * RP + r])
            rrg = plsc.load_gather(rrv, [sv])
            facv[st, pl.ds(g * 16, 16)] = t2g * rrg

    def fire_gather(st):
        pltpu.async_copy(g_hbm.at[gidxv.at[st]], rows.at[st], gsem)

    def wait_gather(st):
        pltpu.make_async_copy(g_hbm.at[gidxv.at[st]], rows.at[st], gsem).wait()

    def scale(st):
        @plsc.parallel_loop(0, B // 16, unroll=2)
        def body(g):
            for b2 in range(16):
                e = g * 16 + b2
                fv = plsc.load_gather(facv.at[st],
                                      [jnp.full((16,), 0, i32) + e])
                rows[st, e, pl.ds(0, 16)] = rows[st, e, pl.ds(0, 16)] * fv

    def fire_scatters(st):
        for j in range(KB):
            pltpu.async_copy(rows.at[st, pl.ds(j * 128, 128), :],
                             out_sp.at[dstv.at[st, j]], ssem, add=True)

    def wait_scatters(st):
        for j in range(KB):
            pltpu.make_async_copy(rows.at[st, pl.ds(j * 128, 128), :],
                                  out_sp.at[dstv.at[st, j]], ssem).wait()

    fire_inputs(0)
    wait_inputs(0)
    index_compute(0)
    fire_gather(0)
    fire_inputs(1)

    def block(blk, _):
        st = blk & 1
        nxt = 1 - st
        wait_gather(st)
        scale(st)
        fire_scatters(st)

        @pl.when((blk > 0) & (blk < NBLK - 1))
        def _():
            wait_scatters(nxt)

        @pl.when(blk < NBLK - 2)
        def _():
            fire_inputs(blk + 2)

        @pl.when(blk < NBLK - 1)
        def _():
            wait_inputs(blk + 1)
            index_compute(nxt)
            fire_gather(nxt)

        return 0

    lax.fori_loop(0, NBLK, block, 0)
    wait_scatters((NBLK - 2) & 1)
    wait_scatters((NBLK - 1) & 1)
    plsc.subcore_barrier()

    @pl.when(sid == 0)
    def _():
        pltpu.sync_copy(out_sp, outp_hbm.at[cid])


def _pass2(s, o, p, t2f, rrec, gflat):
    B = _CB
    return pl.kernel(
        _pass2_body,
        out_type=jax.ShapeDtypeStruct((NC, N, C), f32),
        mesh=_mesh(),
        compiler_params=pltpu.CompilerParams(needs_layout_passes=False, use_tc_tiling_on_sc=False),
        scratch_types=[
            pltpu.VMEM_SHARED((N, C), f32),
            pltpu.VMEM((R * RP,), f32),
            pltpu.VMEM((N,), f32),
            pltpu.VMEM((32000,), f32),
            pltpu.VMEM((2, B), i32),
            pltpu.VMEM((2, B), i32),
            pltpu.VMEM((2, B), i32),
            pltpu.VMEM((2, B), i32),
            pltpu.VMEM((2, B // 128, 128), i32),
            pltpu.VMEM((2, B), f32),
            pltpu.VMEM((2, B, C), f32),
            pltpu.VMEM((125, C), f32),
            pltpu.SemaphoreType.DMA,
            pltpu.SemaphoreType.DMA,
            pltpu.SemaphoreType.DMA,
        ],
    )(s, o, p, t2f, rrec, gflat)


def _final_body(op_ref, b2_ref, out_ref):
    out_ref[...] = op_ref[0] + op_ref[1] + b2_ref[...]


def _final(outp, b2):
    blk = 2000
    return pl.pallas_call(
        _final_body,
        grid=(N // blk,),
        in_specs=[
            pl.BlockSpec((NC, blk, C), lambda i: (0, i, 0)),
            pl.BlockSpec((1, C), lambda i: (0, 0)),
        ],
        out_specs=pl.BlockSpec((blk, C), lambda i: (i, 0)),
        out_shape=jax.ShapeDtypeStruct((N, C), f32),
    )(outp, b2)


def kernel(nhots, hindices, vindices, Wl1, bl1, Wl2, bl2, weights1, weights2,
           bias1, bias2):
    s = hindices[:NT, 0]
    o = vindices[:NT, 1]
    w1flat = weights1.reshape(NSEG, EMB)

    p2d, t1, t2 = _prep(nhots, Wl1, bl1.reshape(1, RP), Wl2, bl2.reshape(1, RP))
    p = p2d.reshape(NT)
    t1f = t1.reshape(R * RP)
    t2f = t2.reshape(R * RP)

    colsum, rowsum = _sums(s, o, p, t1f, t2f)
    w1n, rrec = _norm(w1flat, colsum, rowsum)
    hp = _pass1(s, o, p, t1f, w1n)
    gflat = _gtable(hp, weights2, bias1.reshape(1, EMB)).reshape(NSEG, C)
    outp = _pass2(s, o, p, t2f, rrec.reshape(NSEG), gflat)
    return _final(outp, bias2.reshape(1, C))
